# baseline (speedup 1.0000x reference)
import dataclasses
import functools

import jax
import jax.numpy as jnp
from jax import lax
from jax.experimental import pallas as pl
from jax.experimental.pallas import tpu as pltpu
from jax.experimental.pallas import tpu_sc as plsc

NN = 10000
EE = 320000
FH = 128
NCLS = 10
NGRAPH = 128

NCORE = 2
NSUB = 16
LANES = 16
NWORK = NCORE * NSUB
EDGES_PER_TILE = EE // NWORK
CHUNK = 80
NCHUNK = EDGES_PER_TILE // CHUNK
NBUF = 2
NBLK = 5
NCPB = NCHUNK // NBLK
NROUND = (NCPB - 1) // NBUF
NPAD = 10240
ROWS_PER_TILE = NPAD // NSUB
ZROWS = 40


def _make_agg(depth):
    mesh = plsc.VectorSubcoreMesh(core_axis_name="c", subcore_axis_name="s")

    @functools.partial(
        pl.kernel,
        mesh=mesh,
        out_type=jax.ShapeDtypeStruct((NCORE, NPAD, depth), jnp.float32),
        scratch_types=[
            pltpu.VMEM((NCPB, CHUNK), jnp.int32),
            pltpu.VMEM((NCPB, CHUNK), jnp.int32),
            pltpu.VMEM((NCPB, CHUNK), jnp.int32),
            pltpu.VMEM((NCPB, CHUNK), jnp.int32),
            pltpu.VMEM((CHUNK, depth), jnp.float32),
            pltpu.VMEM((CHUNK, depth), jnp.float32),
            pltpu.VMEM((ZROWS, depth), jnp.float32),
            pltpu.VMEM_SHARED((NPAD, depth), jnp.float32),
            pltpu.SemaphoreType.DMA,
            pltpu.SemaphoreType.DMA,
            pltpu.SemaphoreType.DMA,
            pltpu.SemaphoreType.DMA,
        ],
    )
    def agg(table_hbm, ei_hbm, out_hbm,
            src_a, dst_a, src_b, dst_b, rows0, rows1, zbuf, acc_sh,
            sg0, sg1, ss, sidx):
        cid = lax.axis_index("c")
        sid = lax.axis_index("s")
        wid = sid * NCORE + cid
        rbufs = (rows0, rows1)
        gsems = (sg0, sg1)
        ibufs = ((src_a, dst_a), (src_b, dst_b))
        src_all = ei_hbm.at[0]
        dst_all = ei_hbm.at[1]

        def start_gather(sref, c, buf, sem):
            pltpu.async_copy(table_hbm.at[sref.at[c]], buf, sem)

        def wait_gather(buf, sem):
            pltpu.make_async_copy(table_hbm.at[src_a.at[0]], buf, sem).wait()

        def start_scatter(dref, c, buf):
            return pltpu.async_copy(buf, acc_sh.at[dref.at[c]], ss, add=True)

        pltpu.sync_copy(src_all.at[wid].at[0], src_a)
        pltpu.sync_copy(dst_all.at[wid].at[0], dst_a)
        start_gather(src_a, 0, rows0, sg0)
        start_gather(src_a, 1, rows1, sg1)

        @pl.loop(0, ZROWS)
        def _(r):
            @pl.loop(0, depth // LANES)
            def _(cc):
                zbuf[r, pl.ds(cc * LANES, LANES)] = jnp.zeros(
                    (LANES,), jnp.float32)

        row0 = sid * ROWS_PER_TILE
        for k in range(ROWS_PER_TILE // ZROWS):
            pltpu.async_copy(zbuf, acc_sh.at[pl.ds(row0 + k * ZROWS, ZROWS)],
                             ss)
        for k in range(ROWS_PER_TILE // ZROWS):
            pltpu.make_async_copy(
                zbuf, acc_sh.at[pl.ds(row0, ZROWS)], ss).wait()

        plsc.subcore_barrier()

        for blk in range(NBLK):
            cur_s, cur_d = ibufs[blk % 2]
            if blk + 1 < NBLK:
                nxt_s, nxt_d = ibufs[(blk + 1) % 2]
                pltpu.async_copy(src_all.at[wid].at[blk + 1], nxt_s, sidx)
                pltpu.async_copy(dst_all.at[wid].at[blk + 1], nxt_d, sidx)

            @pl.loop(0, NROUND)
            def _(p, cur_s=cur_s, cur_d=cur_d):
                c0 = p * NBUF
                handles = []
                for b in range(NBUF):
                    wait_gather(rbufs[b], gsems[b])
                    handles.append(start_scatter(cur_d, c0 + b, rbufs[b]))
                for b in range(NBUF):
                    handles[b].wait()

                    @pl.when(c0 + b + NBUF < NCPB)
                    def _(b=b, c0=c0, cur_s=cur_s):
                        start_gather(cur_s, c0 + b + NBUF, rbufs[b],
                                     gsems[b])

            wait_gather(rbufs[0], gsems[0])
            h = start_scatter(cur_d, NCPB - 1, rbufs[0])
            if blk + 1 < NBLK:
                pltpu.make_async_copy(
                    src_all.at[wid].at[0], nxt_s, sidx).wait()
                pltpu.make_async_copy(
                    src_all.at[wid].at[0], nxt_d, sidx).wait()
                start_gather(nxt_s, 1, rbufs[1], gsems[1])
                h.wait()
                start_gather(nxt_s, 0, rbufs[0], gsems[0])
            else:
                h.wait()

        plsc.subcore_barrier()

        pltpu.sync_copy(acc_sh.at[pl.ds(row0, ROWS_PER_TILE)],
                        out_hbm.at[cid].at[pl.ds(row0, ROWS_PER_TILE)])

    return agg


_agg128 = _make_agg(FH)


def _make_deg():
    mesh = plsc.VectorSubcoreMesh(core_axis_name="c", subcore_axis_name="s")
    cp = pltpu.CompilerParams()
    if "needs_layout_passes" in pltpu.CompilerParams.__dataclass_fields__:
        cp = dataclasses.replace(cp, needs_layout_passes=False)

    @functools.partial(
        pl.kernel,
        mesh=mesh,
        compiler_params=cp,
        out_type=jax.ShapeDtypeStruct((NWORK, NPAD), jnp.float32),
        scratch_types=[
            pltpu.VMEM((EDGES_PER_TILE,), jnp.int32),
            pltpu.VMEM((NPAD,), jnp.float32),
        ],
    )
    def deg(ei_hbm, out_hbm, dst_v, hist_v):
        cid = lax.axis_index("c")
        sid = lax.axis_index("s")
        wid = sid * NCORE + cid

        pltpu.sync_copy(ei_hbm.at[1].at[wid], dst_v)

        @pl.loop(0, NPAD // LANES)
        def _(r):
            hist_v[pl.ds(r * LANES, LANES)] = jnp.zeros((LANES,), jnp.float32)

        ones = jnp.ones((LANES,), jnp.float32)

        @pl.loop(0, EDGES_PER_TILE // LANES)
        def _(e):
            idx = dst_v[pl.ds(e * LANES, LANES)]
            plsc.addupdate_scatter(hist_v, [idx], ones)

        pltpu.sync_copy(hist_v, out_hbm.at[wid])

    return deg


_deg = _make_deg()


def _tc_first_body(x_ref, w_ref, degp_ref, hp_ref, dinv_ref):
    ones32 = jnp.ones((NWORK, 1), jnp.float32)
    deg = 1.0 + lax.dot_general(degp_ref[...], ones32,
                                (((0,), (0,)), ((), ())),
                                preferred_element_type=jnp.float32)
    dinv = lax.rsqrt(deg)[:NN]
    h = jnp.dot(x_ref[...], w_ref[...], preferred_element_type=jnp.float32)
    hp_ref[...] = dinv * h
    dinv_ref[...] = dinv


def _tc_first(x, w, degp):
    return pl.pallas_call(
        _tc_first_body,
        out_shape=(
            jax.ShapeDtypeStruct((NN, FH), jnp.float32),
            jax.ShapeDtypeStruct((NN, 1), jnp.float32),
        ),
    )(x, w, degp)


def _tc_mid_body(aggp_ref, hp_ref, dinv_ref, w_ref, b_ref, hpn_ref):
    dinv = dinv_ref[...]
    z = jnp.maximum(
        b_ref[...] + dinv * (aggp_ref[0, :NN] + aggp_ref[1, :NN] + hp_ref[...]),
        0.0)
    h = jnp.dot(z, w_ref[...], preferred_element_type=jnp.float32)
    hpn_ref[...] = dinv * h


def _tc_mid(aggp, hp, dinv, w, b):
    dout = w.shape[1]
    return pl.pallas_call(
        _tc_mid_body,
        out_shape=jax.ShapeDtypeStruct((NN, dout), jnp.float32),
    )(aggp, hp, dinv, w, b.reshape(1, b.shape[0]))


def _tc_final_body(aggp_ref, hp_ref, dinv_ref, b_ref, batch_ref, out_ref):
    z = b_ref[...] + dinv_ref[...] * (
        aggp_ref[0, :NN] + aggp_ref[1, :NN] + hp_ref[...])
    gids = lax.broadcasted_iota(jnp.int32, (NGRAPH, NN), 0)
    onehot = (gids == batch_ref[...]).astype(jnp.float32)
    sums = jnp.dot(onehot, z, preferred_element_type=jnp.float32)
    counts = jnp.sum(onehot, axis=1, keepdims=True)
    pooled = sums[:, :NCLS] / jnp.maximum(counts, 1.0)
    m = jnp.max(pooled, axis=1, keepdims=True)
    shifted = pooled - m
    lse = jnp.log(jnp.sum(jnp.exp(shifted), axis=1, keepdims=True))
    out_ref[...] = shifted - lse


def _tc_final(aggp, hp, dinv, b, batch2d):
    return pl.pallas_call(
        _tc_final_body,
        out_shape=jax.ShapeDtypeStruct((NGRAPH, NCLS), jnp.float32),
    )(aggp, hp, dinv, b.reshape(1, b.shape[0]), batch2d)


def kernel(x, edge_index, batch, W1, b1, W2, b2, W3, b3, W4, b4, W5, b5):
    ei = edge_index.reshape(2, NWORK, NBLK, NCPB, CHUNK)
    eid = edge_index.reshape(2, NWORK, EDGES_PER_TILE)

    degp = _deg(eid)

    hp, dinv = _tc_first(x, W1, degp)
    aggp = _agg128(hp, ei)
    hp = _tc_mid(aggp, hp, dinv, W2, b1)
    aggp = _agg128(hp, ei)
    hp = _tc_mid(aggp, hp, dinv, W3, b2)
    aggp = _agg128(hp, ei)
    hp = _tc_mid(aggp, hp, dinv, W4, b3)
    aggp = _agg128(hp, ei)

    w5p = jnp.pad(W5, ((0, 0), (0, FH - NCLS)))
    b5p = jnp.pad(b5, (0, FH - NCLS))
    hp = _tc_mid(aggp, hp, dinv, w5p, b4)
    aggp = _agg128(hp, ei)

    return _tc_final(aggp, hp, dinv, b5p, batch.reshape(1, NN))


# speedup vs baseline: 1.0188x; 1.0188x over previous
import dataclasses
import functools

import jax
import jax.numpy as jnp
from jax import lax
from jax.experimental import pallas as pl
from jax.experimental.pallas import tpu as pltpu
from jax.experimental.pallas import tpu_sc as plsc

NN = 10000
EE = 320000
FH = 128
NCLS = 10
NGRAPH = 128

NCORE = 2
NSUB = 16
LANES = 16
NWORK = NCORE * NSUB
EDGES_PER_TILE = EE // NWORK
CHUNK = 80
NCHUNK = EDGES_PER_TILE // CHUNK
NBUF = 3
NBLK = 5
NCPB = NCHUNK // NBLK
NFULLB = NCPB // NBUF
NTAILB = NCPB - NBUF * NFULLB
NPAD = 10240
ROWS_PER_TILE = NPAD // NSUB


def _make_agg(depth):
    mesh = plsc.VectorSubcoreMesh(core_axis_name="c", subcore_axis_name="s")

    @functools.partial(
        pl.kernel,
        mesh=mesh,
        out_type=jax.ShapeDtypeStruct((NCORE, NPAD, depth), jnp.float32),
        scratch_types=[
            pltpu.VMEM((NCPB, CHUNK), jnp.int32),
            pltpu.VMEM((NCPB, CHUNK), jnp.int32),
            pltpu.VMEM((CHUNK, depth), jnp.float32),
            pltpu.VMEM((CHUNK, depth), jnp.float32),
            pltpu.VMEM((CHUNK, depth), jnp.float32),
            pltpu.VMEM_SHARED((NPAD, depth), jnp.float32),
            pltpu.SemaphoreType.DMA,
            pltpu.SemaphoreType.DMA,
            pltpu.SemaphoreType.DMA,
            pltpu.SemaphoreType.DMA,
        ],
    )
    def agg(table_hbm, ei_hbm, out_hbm,
            src_v, dst_v, rows0, rows1, rows2, acc_sh,
            sg0, sg1, sg2, ss):
        cid = lax.axis_index("c")
        sid = lax.axis_index("s")
        wid = sid * NCORE + cid
        bufs = (rows0, rows1, rows2)
        gsems = (sg0, sg1, sg2)
        src_all = ei_hbm.at[0]
        dst_all = ei_hbm.at[1]

        def start_gather(c, buf, sem):
            pltpu.async_copy(table_hbm.at[src_v.at[c]], buf, sem)

        def wait_gather(buf, sem):
            pltpu.make_async_copy(table_hbm.at[src_v.at[0]], buf, sem).wait()

        pltpu.sync_copy(src_all.at[wid].at[0], src_v)
        pltpu.sync_copy(dst_all.at[wid].at[0], dst_v)
        start_gather(1, bufs[1], gsems[1])
        start_gather(2, bufs[2], gsems[2])

        @pl.loop(0, CHUNK)
        def _(r):
            @pl.loop(0, depth // LANES)
            def _(cc):
                rows0[r, pl.ds(cc * LANES, LANES)] = jnp.zeros(
                    (LANES,), jnp.float32)

        row0 = sid * ROWS_PER_TILE
        for k in range(ROWS_PER_TILE // CHUNK):
            pltpu.async_copy(rows0, acc_sh.at[pl.ds(row0 + k * CHUNK, CHUNK)],
                             ss)
        for k in range(ROWS_PER_TILE // CHUNK):
            pltpu.make_async_copy(
                rows0, acc_sh.at[pl.ds(row0, CHUNK)], ss).wait()
        start_gather(0, bufs[0], gsems[0])

        plsc.subcore_barrier()

        @pl.loop(0, NBLK)
        def _(blk):
            @pl.when(blk > 0)
            def _():
                pltpu.sync_copy(src_all.at[wid].at[blk], src_v)
                pltpu.sync_copy(dst_all.at[wid].at[blk], dst_v)

                for b in range(NBUF):
                    start_gather(b, bufs[b], gsems[b])

            @pl.loop(0, NFULLB)
            def _(p):
                c0 = p * NBUF
                handles = []
                for b in range(NBUF):
                    wait_gather(bufs[b], gsems[b])
                    handles.append(
                        pltpu.async_copy(bufs[b], acc_sh.at[dst_v.at[c0 + b]],
                                         ss, add=True))
                for b in range(NBUF):
                    handles[b].wait()

                    @pl.when(c0 + b + NBUF < NCPB)
                    def _(b=b, c0=c0):
                        start_gather(c0 + b + NBUF, bufs[b], gsems[b])

            for b in range(NTAILB):
                wait_gather(bufs[b], gsems[b])
                pltpu.sync_copy(bufs[b], acc_sh.at[dst_v.at[NBUF * NFULLB + b]],
                                add=True)

        plsc.subcore_barrier()

        pltpu.sync_copy(acc_sh.at[pl.ds(row0, ROWS_PER_TILE)],
                        out_hbm.at[cid].at[pl.ds(row0, ROWS_PER_TILE)])

    return agg


_agg128 = _make_agg(FH)


def _make_deg():
    mesh = plsc.VectorSubcoreMesh(core_axis_name="c", subcore_axis_name="s")
    cp = pltpu.CompilerParams()
    if "needs_layout_passes" in pltpu.CompilerParams.__dataclass_fields__:
        cp = dataclasses.replace(cp, needs_layout_passes=False)

    @functools.partial(
        pl.kernel,
        mesh=mesh,
        compiler_params=cp,
        out_type=jax.ShapeDtypeStruct((NWORK, NPAD), jnp.float32),
        scratch_types=[
            pltpu.VMEM((EDGES_PER_TILE,), jnp.int32),
            pltpu.VMEM((NPAD,), jnp.float32),
        ],
    )
    def deg(ei_hbm, out_hbm, dst_v, hist_v):
        cid = lax.axis_index("c")
        sid = lax.axis_index("s")
        wid = sid * NCORE + cid

        pltpu.sync_copy(ei_hbm.at[1].at[wid], dst_v)

        @pl.loop(0, NPAD // LANES)
        def _(r):
            hist_v[pl.ds(r * LANES, LANES)] = jnp.zeros((LANES,), jnp.float32)

        ones = jnp.ones((LANES,), jnp.float32)

        @pl.loop(0, EDGES_PER_TILE // LANES)
        def _(e):
            idx = dst_v[pl.ds(e * LANES, LANES)]
            plsc.addupdate_scatter(hist_v, [idx], ones)

        pltpu.sync_copy(hist_v, out_hbm.at[wid])

    return deg


_deg = _make_deg()


def _tc_first_body(x_ref, w_ref, degp_ref, hp_ref, dinv_ref):
    ones32 = jnp.ones((NWORK, 1), jnp.float32)
    deg = 1.0 + lax.dot_general(degp_ref[...], ones32,
                                (((0,), (0,)), ((), ())),
                                preferred_element_type=jnp.float32)
    dinv = lax.rsqrt(deg)[:NN]
    h = jnp.dot(x_ref[...], w_ref[...], preferred_element_type=jnp.float32)
    hp_ref[...] = dinv * h
    dinv_ref[...] = dinv


def _tc_first(x, w, degp):
    return pl.pallas_call(
        _tc_first_body,
        out_shape=(
            jax.ShapeDtypeStruct((NN, FH), jnp.float32),
            jax.ShapeDtypeStruct((NN, 1), jnp.float32),
        ),
    )(x, w, degp)


def _tc_mid_body(aggp_ref, hp_ref, dinv_ref, w_ref, b_ref, hpn_ref):
    dinv = dinv_ref[...]
    z = jnp.maximum(
        b_ref[...] + dinv * (aggp_ref[0, :NN] + aggp_ref[1, :NN] + hp_ref[...]),
        0.0)
    h = jnp.dot(z, w_ref[...], preferred_element_type=jnp.float32)
    hpn_ref[...] = dinv * h


def _tc_mid(aggp, hp, dinv, w, b):
    dout = w.shape[1]
    return pl.pallas_call(
        _tc_mid_body,
        out_shape=jax.ShapeDtypeStruct((NN, dout), jnp.float32),
    )(aggp, hp, dinv, w, b.reshape(1, b.shape[0]))


def _tc_final_body(aggp_ref, hp_ref, dinv_ref, b_ref, batch_ref, out_ref):
    z = b_ref[...] + dinv_ref[...] * (
        aggp_ref[0, :NN] + aggp_ref[1, :NN] + hp_ref[...])
    gids = lax.broadcasted_iota(jnp.int32, (NGRAPH, NN), 0)
    onehot = (gids == batch_ref[...]).astype(jnp.float32)
    sums = jnp.dot(onehot, z, preferred_element_type=jnp.float32)
    counts = jnp.sum(onehot, axis=1, keepdims=True)
    pooled = sums[:, :NCLS] / jnp.maximum(counts, 1.0)
    m = jnp.max(pooled, axis=1, keepdims=True)
    shifted = pooled - m
    lse = jnp.log(jnp.sum(jnp.exp(shifted), axis=1, keepdims=True))
    out_ref[...] = shifted - lse


def _tc_final(aggp, hp, dinv, b, batch2d):
    return pl.pallas_call(
        _tc_final_body,
        out_shape=jax.ShapeDtypeStruct((NGRAPH, NCLS), jnp.float32),
    )(aggp, hp, dinv, b.reshape(1, b.shape[0]), batch2d)


def kernel(x, edge_index, batch, W1, b1, W2, b2, W3, b3, W4, b4, W5, b5):
    ei = edge_index.reshape(2, NWORK, NBLK, NCPB, CHUNK)
    eid = edge_index.reshape(2, NWORK, EDGES_PER_TILE)

    degp = _deg(eid)

    hp, dinv = _tc_first(x, W1, degp)
    aggp = _agg128(hp, ei)
    hp = _tc_mid(aggp, hp, dinv, W2, b1)
    aggp = _agg128(hp, ei)
    hp = _tc_mid(aggp, hp, dinv, W3, b2)
    aggp = _agg128(hp, ei)
    hp = _tc_mid(aggp, hp, dinv, W4, b3)
    aggp = _agg128(hp, ei)

    w5p = jnp.pad(W5, ((0, 0), (0, FH - NCLS)))
    b5p = jnp.pad(b5, (0, FH - NCLS))
    hp = _tc_mid(aggp, hp, dinv, w5p, b4)
    aggp = _agg128(hp, ei)

    return _tc_final(aggp, hp, dinv, b5p, batch.reshape(1, NN))
